# baseline (speedup 1.0000x reference)
import functools

import jax
import jax.numpy as jnp
from jax import lax
from jax.experimental import pallas as pl
from jax.experimental.pallas import tpu as pltpu
from jax.experimental.pallas import tpu_sc as plsc

_N = 10000
_E = 320000
_D = 128
_H = 64
_O = 4
_R = 3

_NPAD = 10240
_K = 128
_ER = _E // _K
_ERP = 2560
_BLK = 1024
_NB = _NPAD // _BLK
_HW = _H // 2
_W2 = 2 * _O

_SR1 = 4 * _NPAD
_SA1 = _SR1 + 128
_ST1 = _SA1 // 16

_L2W = 16
_SR2 = 8 * _NPAD
_SA2 = _SR2 + 128
_ST2 = _SA2 // 16

_NCH1 = _ERP // 16
_PH1 = _NCH1 // 4
_NCH2 = _ERP // 32

_sc_params = pltpu.CompilerParams(use_tc_tiling_on_sc=False)


def _mm_body(x_ref, w_ref, b_ref, o_ref):
    o_ref[...] = (
        jnp.dot(x_ref[...], w_ref[...], preferred_element_type=jnp.float32,
                precision=lax.Precision.HIGHEST)
        + b_ref[...]
    )


def _edge_prep_body(ei_ref, t_ref, s4c_ref, s16_ref, x1_ref, xc_ref, x2_ref,
                    tt_ref):
    i = pl.program_id(0)
    rows = _ERP // 10
    rid = jax.lax.broadcasted_iota(jnp.int32, (rows, _K), 0) + i * rows
    live = rid < _ER
    s = ei_ref[0]
    d = ei_ref[1]
    t = t_ref[...]
    s4 = jnp.where(live, s * 4, 0)
    s4c_ref[0] = s4
    s4c_ref[1] = s4 + 1
    s16_ref[...] = jnp.where(live, s * 8, 0)
    x1_ref[...] = jnp.where(live, d * 4 + t, _SR1)
    xc_ref[...] = jnp.where(live, d * 4 + 3, _SR1 + 64)
    x2_ref[...] = jnp.where(live, d * 8 + t, _SR2)
    tt_ref[...] = jnp.where(live, t, 3)


def _seg1_body(s4c_hbm, x1_hbm, xc_hbm, tt_hbm, y_hbm, za_hbm, oh_hbm,
               aout_hbm,
               a_sh, src_v, sidx_v, sidxc_v, tt_v, rows0_v, rows1_v, cnt_v,
               sem0, sem1):
    cid = lax.axis_index("c")
    sid = lax.axis_index("s")
    base = sid * _ST1

    pltpu.sync_copy(za_hbm, a_sh.at[pl.ds(base, _ST1)])

    for p in range(4):
        row0 = sid * _NCH1 + p * _PH1
        pltpu.sync_copy(s4c_hbm.at[cid, pl.ds(row0, _PH1)], src_v)
        pltpu.sync_copy(x1_hbm.at[pl.ds(row0, _PH1)], sidx_v)
        pltpu.sync_copy(xc_hbm.at[pl.ds(row0, _PH1)], sidxc_v)
        pltpu.sync_copy(tt_hbm.at[pl.ds(row0, _PH1)], tt_v)
        if p == 0:
            plsc.subcore_barrier()

        do_counts = cid == p // 2

        def gather(ci, buf, sem):
            pltpu.async_copy(y_hbm.at[src_v.at[ci]], buf, sem)

        def drain_scatter(ci, buf, sem):
            pltpu.make_async_copy(y_hbm.at[src_v.at[ci]], buf, sem).wait()
            pltpu.sync_copy(buf, a_sh.at[sidx_v.at[ci]], add=True)

            @pl.when(do_counts)
            def _():
                pltpu.sync_copy(oh_hbm.at[tt_v.at[ci]], cnt_v)
                pltpu.sync_copy(cnt_v, a_sh.at[sidxc_v.at[ci]], add=True)

        gather(0, rows0_v, sem0)

        def pair(g, _):
            c0 = 2 * g
            gather(c0 + 1, rows1_v, sem1)
            drain_scatter(c0, rows0_v, sem0)

            @pl.when(c0 + 2 < _PH1)
            def _():
                gather(c0 + 2, rows0_v, sem0)
            drain_scatter(c0 + 1, rows1_v, sem1)
            return 0
        lax.fori_loop(0, _PH1 // 2, pair, 0)

    plsc.subcore_barrier()
    pltpu.sync_copy(a_sh.at[pl.ds(base, _ST1)],
                    aout_hbm.at[cid, pl.ds(base, _ST1)])


def _seg2_body(s16_hbm, x2_hbm, y_hbm, za_hbm, aout_hbm,
               a_sh, src_v, sidx_v, rows0_v, rows1_v, sem0, sem1):
    cid = lax.axis_index("c")
    sid = lax.axis_index("s")
    wid = sid * 2 + cid
    base = sid * _ST2

    pltpu.sync_copy(za_hbm, a_sh.at[pl.ds(base, _ST2)])
    pltpu.sync_copy(s16_hbm.at[pl.ds(wid * _NCH2, _NCH2)], src_v)
    pltpu.sync_copy(x2_hbm.at[pl.ds(wid * _NCH2, _NCH2)], sidx_v)
    plsc.subcore_barrier()

    def gather(ci, buf, sem):
        pltpu.async_copy(y_hbm.at[src_v.at[ci]], buf, sem)

    def drain_scatter(ci, buf, sem):
        pltpu.make_async_copy(y_hbm.at[src_v.at[ci]], buf, sem).wait()
        pltpu.sync_copy(buf, a_sh.at[sidx_v.at[ci]], add=True)

    gather(0, rows0_v, sem0)

    def pair(g, _):
        c0 = 2 * g
        gather(c0 + 1, rows1_v, sem1)
        drain_scatter(c0, rows0_v, sem0)

        @pl.when(c0 + 2 < _NCH2)
        def _():
            gather(c0 + 2, rows0_v, sem0)
        drain_scatter(c0 + 1, rows1_v, sem1)
        return 0
    lax.fori_loop(0, _NCH2 // 2, pair, 0)

    plsc.subcore_barrier()
    pltpu.sync_copy(a_sh.at[pl.ds(base, _ST2)],
                    aout_hbm.at[cid, pl.ds(base, _ST2)])


@functools.cache
def _get_seg1():
    mesh = plsc.VectorSubcoreMesh(core_axis_name="c", subcore_axis_name="s")
    return pl.kernel(
        _seg1_body,
        out_type=jax.ShapeDtypeStruct((2, _SA1, _HW), jnp.float32),
        mesh=mesh,
        scratch_types=[
            pltpu.VMEM_SHARED((_SA1, _HW), jnp.float32),
            pltpu.VMEM((_PH1, _K), jnp.int32),
            pltpu.VMEM((_PH1, _K), jnp.int32),
            pltpu.VMEM((_PH1, _K), jnp.int32),
            pltpu.VMEM((_PH1, _K), jnp.int32),
            pltpu.VMEM((_K, _HW), jnp.float32),
            pltpu.VMEM((_K, _HW), jnp.float32),
            pltpu.VMEM((_K, _HW), jnp.float32),
            pltpu.SemaphoreType.DMA,
            pltpu.SemaphoreType.DMA,
        ],
        compiler_params=_sc_params,
    )


@functools.cache
def _get_seg2():
    mesh = plsc.VectorSubcoreMesh(core_axis_name="c", subcore_axis_name="s")
    return pl.kernel(
        _seg2_body,
        out_type=jax.ShapeDtypeStruct((2, _SA2, _L2W), jnp.float32),
        mesh=mesh,
        scratch_types=[
            pltpu.VMEM_SHARED((_SA2, _L2W), jnp.float32),
            pltpu.VMEM((_NCH2, _K), jnp.int32),
            pltpu.VMEM((_NCH2, _K), jnp.int32),
            pltpu.VMEM((_K, _L2W), jnp.float32),
            pltpu.VMEM((_K, _L2W), jnp.float32),
            pltpu.SemaphoreType.DMA,
            pltpu.SemaphoreType.DMA,
        ],
        compiler_params=_sc_params,
    )


def _comb1_body(y_ref, a_ref, comp_ref, w_ref, b_ref, o_ref):
    acc = y_ref[:, _H:]
    for r in range(_R):
        cnt = (a_ref[0, :, 96 + 8 * r:97 + 8 * r]
               + a_ref[1, :, 96 + 8 * r:97 + 8 * r])
        inv = 1.0 / jnp.maximum(cnt, 1.0)
        s = jnp.concatenate(
            [a_ref[0, :, 32 * r:32 * r + 32], a_ref[1, :, 32 * r:32 * r + 32]],
            axis=1)
        acc = acc + s * (comp_ref[r, 0] * inv)
    h = jnp.maximum(acc, 0.0)
    y2 = (
        jnp.dot(h, w_ref[...], preferred_element_type=jnp.float32,
                precision=lax.Precision.HIGHEST)
        + b_ref[...]
    )
    o_ref[...] = jnp.concatenate(
        [y2, jnp.zeros((_BLK, _D - _W2), jnp.float32)], axis=1)


def _comb2_body(y_ref, a_ref, ac_ref, comp_ref, o_ref):
    acc = y_ref[:, _O:_W2]
    for r in range(_R):
        cnt = (ac_ref[0, :, 96 + 8 * r:97 + 8 * r]
               + ac_ref[1, :, 96 + 8 * r:97 + 8 * r])
        inv = 1.0 / jnp.maximum(cnt, 1.0)
        s = (a_ref[0, :, 16 * r:16 * r + _O]
             + a_ref[1, :, 16 * r:16 * r + _O])
        acc = acc + s * (comp_ref[r, 0] * inv)
    z = acc - jnp.max(acc, axis=1, keepdims=True)
    ez = jnp.exp(z)
    o_ref[...] = ez / jnp.sum(ez, axis=1, keepdims=True)


def kernel(x, edge_index, edge_type, basis1, comp1, root1, bias1,
           basis2, comp2, root2, bias2):
    ei3 = edge_index.astype(jnp.int32).reshape(2, _ER, _K)
    t2 = edge_type.astype(jnp.int32).reshape(_ER, _K)

    erows = _ERP // 10
    s4c, s16, x1, xc, x2, tt = pl.pallas_call(
        _edge_prep_body,
        grid=(10,),
        in_specs=[
            pl.BlockSpec((2, erows, _K), lambda i: (0, i, 0)),
            pl.BlockSpec((erows, _K), lambda i: (i, 0)),
        ],
        out_specs=[
            pl.BlockSpec((2, erows, _K), lambda i: (0, i, 0)),
        ] + [pl.BlockSpec((erows, _K), lambda i: (i, 0))] * 5,
        out_shape=[
            jax.ShapeDtypeStruct((2, _ERP, _K), jnp.int32),
        ] + [jax.ShapeDtypeStruct((_ERP, _K), jnp.int32)] * 5,
    )(ei3, t2)

    w1cat = jnp.concatenate([basis1[0], root1], axis=1)
    b1cat = jnp.concatenate([jnp.zeros((_H,), jnp.float32), bias1])[None, :]
    y1r = pl.pallas_call(
        _mm_body,
        grid=(10,),
        in_specs=[
            pl.BlockSpec((1000, _D), lambda i: (i, 0)),
            pl.BlockSpec((_D, _D), lambda i: (0, 0)),
            pl.BlockSpec((1, _D), lambda i: (0, 0)),
        ],
        out_specs=pl.BlockSpec((1000, _D), lambda i: (i, 0)),
        out_shape=jax.ShapeDtypeStruct((_N, _D), jnp.float32),
    )(x, w1cat, b1cat)

    za1 = jnp.zeros((_ST1, _HW), jnp.float32)
    lane = jnp.arange(_HW)[None, :]
    row = jnp.arange(8)[:, None]
    onehot = ((lane >= 8 * row) & (lane < 8 * row + 8)
              & (row < _R)).astype(jnp.float32)
    a1p = _get_seg1()(s4c, x1, xc, tt, y1r.reshape(4 * _N, _HW), za1, onehot)
    a1v = a1p.reshape(2, _SA1 // 4, _D)

    w2cat = jnp.concatenate([basis2[0], root2], axis=1)
    b2cat = jnp.concatenate([jnp.zeros((_O,), jnp.float32), bias2])[None, :]
    y2p = pl.pallas_call(
        _comb1_body,
        grid=(_NB,),
        in_specs=[
            pl.BlockSpec((_BLK, _D), lambda i: (i, 0)),
            pl.BlockSpec((2, _BLK, _D), lambda i: (0, i, 0)),
            pl.BlockSpec((_R, 1), lambda i: (0, 0)),
            pl.BlockSpec((_H, _W2), lambda i: (0, 0)),
            pl.BlockSpec((1, _W2), lambda i: (0, 0)),
        ],
        out_specs=pl.BlockSpec((_BLK, _D), lambda i: (i, 0)),
        out_shape=jax.ShapeDtypeStruct((_N, _D), jnp.float32),
    )(y1r, a1v, comp1, w2cat, b2cat)

    za2 = jnp.zeros((_ST2, _L2W), jnp.float32)
    a2p = _get_seg2()(s16, x2, y2p.reshape(8 * _N, _L2W), za2)
    a2v = a2p.reshape(2, _SA2 // 8, _D)

    out = pl.pallas_call(
        _comb2_body,
        grid=(_NB,),
        in_specs=[
            pl.BlockSpec((_BLK, _D), lambda i: (i, 0)),
            pl.BlockSpec((2, _BLK, _D), lambda i: (0, i, 0)),
            pl.BlockSpec((2, _BLK, _D), lambda i: (0, i, 0)),
            pl.BlockSpec((_R, 1), lambda i: (0, 0)),
        ],
        out_specs=pl.BlockSpec((_BLK, _O), lambda i: (i, 0)),
        out_shape=jax.ShapeDtypeStruct((_N, _O), jnp.float32),
    )(y2p, a2v, a1v, comp2)

    return out


# speedup vs baseline: 7.8369x; 7.8369x over previous
import functools

import jax
import jax.numpy as jnp
from jax import lax
from jax.experimental import pallas as pl
from jax.experimental.pallas import tpu as pltpu
from jax.experimental.pallas import tpu_sc as plsc

_N = 10000
_E = 320000
_D = 128
_H = 64
_O = 4
_R = 3

_NPAD = 10240
_K = 128
_ER = _E // _K
_ERP = 2560
_BLK = 1024
_NB = _NPAD // _BLK
_HW = _H // 2
_W2 = 2 * _O

_SR1 = 4 * _NPAD
_SA1 = _SR1 + 128
_ST1 = _SA1 // 16

_L2W = 16
_SR2 = 8 * _NPAD
_SA2 = _SR2 + 128
_ST2 = _SA2 // 16

_NCH1 = _ERP // 16
_PH1 = _NCH1 // 4
_NCH2 = _ERP // 32

_sc_params = pltpu.CompilerParams(use_tc_tiling_on_sc=False)


def _mm_body(x_ref, w_ref, b_ref, o_ref):
    o_ref[...] = (
        jnp.dot(x_ref[...], w_ref[...], preferred_element_type=jnp.float32,
                precision=lax.Precision.HIGHEST)
        + b_ref[...]
    )


def _edge_prep_body(ei_ref, t_ref, s4c_ref, s16_ref, x1_ref, x2_ref,
                    xc0_ref, xc1_ref, xc2_ref):
    i = pl.program_id(0)
    rows = _ERP // 10
    rid = jax.lax.broadcasted_iota(jnp.int32, (rows, _K), 0) + i * rows
    lane = jax.lax.broadcasted_iota(jnp.int32, (rows, _K), 1)
    live = rid < _ER
    s = ei_ref[0]
    d = ei_ref[1]
    t = t_ref[...]
    s4 = jnp.where(live, s * 4, 0)
    s4c_ref[0] = s4
    s4c_ref[1] = s4 + 1
    s16_ref[...] = jnp.where(live, s * 8, 0)
    x1_ref[...] = jnp.where(live, d * 4 + t, _SR1)
    x2_ref[...] = jnp.where(live, d * 8 + t, _SR2)
    garb = _SR1 + ((rid + lane) % 128)
    for r, ref in enumerate((xc0_ref, xc1_ref, xc2_ref)):
        ref[...] = jnp.where(live & (t == r), d * 4 + 3, garb)


def _seg1_body(s4c_hbm, x1_hbm, xc0_hbm, xc1_hbm, xc2_hbm, y_hbm, za_hbm,
               pat_hbm, aout_hbm,
               a_sh, src_v, sidx_v, xc0_v, xc1_v, xc2_v, rows0_v, rows1_v,
               pat0_v, pat1_v, pat2_v, sem0, sem1, semc):
    cid = lax.axis_index("c")
    sid = lax.axis_index("s")
    base = sid * _ST1

    pltpu.sync_copy(za_hbm, a_sh.at[pl.ds(base, _ST1)])
    pltpu.sync_copy(pat_hbm.at[pl.ds(0, _K)], pat0_v)
    pltpu.sync_copy(pat_hbm.at[pl.ds(_K, _K)], pat1_v)
    pltpu.sync_copy(pat_hbm.at[pl.ds(2 * _K, _K)], pat2_v)

    for p in range(4):
        row0 = sid * _NCH1 + p * _PH1
        pltpu.sync_copy(s4c_hbm.at[cid, pl.ds(row0, _PH1)], src_v)
        pltpu.sync_copy(x1_hbm.at[pl.ds(row0, _PH1)], sidx_v)
        pltpu.sync_copy(xc0_hbm.at[pl.ds(row0, _PH1)], xc0_v)
        pltpu.sync_copy(xc1_hbm.at[pl.ds(row0, _PH1)], xc1_v)
        pltpu.sync_copy(xc2_hbm.at[pl.ds(row0, _PH1)], xc2_v)
        if p == 0:
            plsc.subcore_barrier()

        do_counts = cid == p // 2

        def gather(ci, buf, sem):
            pltpu.async_copy(y_hbm.at[src_v.at[ci]], buf, sem)

        def drain_scatter(ci, buf, sem):
            pltpu.make_async_copy(y_hbm.at[src_v.at[ci]], buf, sem).wait()
            pltpu.sync_copy(buf, a_sh.at[sidx_v.at[ci]], add=True)

            @pl.when(do_counts)
            def _():
                pltpu.async_copy(pat0_v, a_sh.at[xc0_v.at[ci]], semc,
                                 add=True)
                pltpu.async_copy(pat1_v, a_sh.at[xc1_v.at[ci]], semc,
                                 add=True)
                pltpu.async_copy(pat2_v, a_sh.at[xc2_v.at[ci]], semc,
                                 add=True)

        gather(0, rows0_v, sem0)

        def pair(g, _):
            c0 = 2 * g
            gather(c0 + 1, rows1_v, sem1)
            drain_scatter(c0, rows0_v, sem0)

            @pl.when(c0 + 2 < _PH1)
            def _():
                gather(c0 + 2, rows0_v, sem0)
            drain_scatter(c0 + 1, rows1_v, sem1)
            return 0
        lax.fori_loop(0, _PH1 // 2, pair, 0)

        @pl.when(do_counts)
        def _():
            def drain_c(i, _):
                pltpu.make_async_copy(
                    pat0_v, a_sh.at[xc0_v.at[0]], semc).wait()
                return 0
            lax.fori_loop(0, 3 * _PH1, drain_c, 0)

    plsc.subcore_barrier()
    pltpu.sync_copy(a_sh.at[pl.ds(base, _ST1)],
                    aout_hbm.at[cid, pl.ds(base, _ST1)])


def _seg2_body(s16_hbm, x2_hbm, y_hbm, za_hbm, aout_hbm,
               a_sh, src_v, sidx_v, rows0_v, rows1_v, sem0, sem1):
    cid = lax.axis_index("c")
    sid = lax.axis_index("s")
    wid = sid * 2 + cid
    base = sid * _ST2

    pltpu.sync_copy(za_hbm, a_sh.at[pl.ds(base, _ST2)])
    pltpu.sync_copy(s16_hbm.at[pl.ds(wid * _NCH2, _NCH2)], src_v)
    pltpu.sync_copy(x2_hbm.at[pl.ds(wid * _NCH2, _NCH2)], sidx_v)
    plsc.subcore_barrier()

    def gather(ci, buf, sem):
        pltpu.async_copy(y_hbm.at[src_v.at[ci]], buf, sem)

    def drain_scatter(ci, buf, sem):
        pltpu.make_async_copy(y_hbm.at[src_v.at[ci]], buf, sem).wait()
        pltpu.sync_copy(buf, a_sh.at[sidx_v.at[ci]], add=True)

    gather(0, rows0_v, sem0)

    def pair(g, _):
        c0 = 2 * g
        gather(c0 + 1, rows1_v, sem1)
        drain_scatter(c0, rows0_v, sem0)

        @pl.when(c0 + 2 < _NCH2)
        def _():
            gather(c0 + 2, rows0_v, sem0)
        drain_scatter(c0 + 1, rows1_v, sem1)
        return 0
    lax.fori_loop(0, _NCH2 // 2, pair, 0)

    plsc.subcore_barrier()
    pltpu.sync_copy(a_sh.at[pl.ds(base, _ST2)],
                    aout_hbm.at[cid, pl.ds(base, _ST2)])


@functools.cache
def _get_seg1():
    mesh = plsc.VectorSubcoreMesh(core_axis_name="c", subcore_axis_name="s")
    return pl.kernel(
        _seg1_body,
        out_type=jax.ShapeDtypeStruct((2, _SA1, _HW), jnp.float32),
        mesh=mesh,
        scratch_types=[
            pltpu.VMEM_SHARED((_SA1, _HW), jnp.float32),
            pltpu.VMEM((_PH1, _K), jnp.int32),
            pltpu.VMEM((_PH1, _K), jnp.int32),
            pltpu.VMEM((_PH1, _K), jnp.int32),
            pltpu.VMEM((_PH1, _K), jnp.int32),
            pltpu.VMEM((_PH1, _K), jnp.int32),
            pltpu.VMEM((_K, _HW), jnp.float32),
            pltpu.VMEM((_K, _HW), jnp.float32),
            pltpu.VMEM((_K, _HW), jnp.float32),
            pltpu.VMEM((_K, _HW), jnp.float32),
            pltpu.VMEM((_K, _HW), jnp.float32),
            pltpu.SemaphoreType.DMA,
            pltpu.SemaphoreType.DMA,
            pltpu.SemaphoreType.DMA,
        ],
        compiler_params=_sc_params,
    )


@functools.cache
def _get_seg2():
    mesh = plsc.VectorSubcoreMesh(core_axis_name="c", subcore_axis_name="s")
    return pl.kernel(
        _seg2_body,
        out_type=jax.ShapeDtypeStruct((2, _SA2, _L2W), jnp.float32),
        mesh=mesh,
        scratch_types=[
            pltpu.VMEM_SHARED((_SA2, _L2W), jnp.float32),
            pltpu.VMEM((_NCH2, _K), jnp.int32),
            pltpu.VMEM((_NCH2, _K), jnp.int32),
            pltpu.VMEM((_K, _L2W), jnp.float32),
            pltpu.VMEM((_K, _L2W), jnp.float32),
            pltpu.SemaphoreType.DMA,
            pltpu.SemaphoreType.DMA,
        ],
        compiler_params=_sc_params,
    )


def _comb1_body(y_ref, a_ref, comp_ref, w_ref, b_ref, o_ref):
    acc = y_ref[:, _H:]
    for r in range(_R):
        cnt = (a_ref[0, :, 96 + 8 * r:97 + 8 * r]
               + a_ref[1, :, 96 + 8 * r:97 + 8 * r])
        inv = 1.0 / jnp.maximum(cnt, 1.0)
        s = jnp.concatenate(
            [a_ref[0, :, 32 * r:32 * r + 32], a_ref[1, :, 32 * r:32 * r + 32]],
            axis=1)
        acc = acc + s * (comp_ref[r, 0] * inv)
    h = jnp.maximum(acc, 0.0)
    y2 = (
        jnp.dot(h, w_ref[...], preferred_element_type=jnp.float32,
                precision=lax.Precision.HIGHEST)
        + b_ref[...]
    )
    o_ref[...] = jnp.concatenate(
        [y2, jnp.zeros((_BLK, _D - _W2), jnp.float32)], axis=1)


def _comb2_body(y_ref, a_ref, ac_ref, comp_ref, o_ref):
    acc = y_ref[:, _O:_W2]
    for r in range(_R):
        cnt = (ac_ref[0, :, 96 + 8 * r:97 + 8 * r]
               + ac_ref[1, :, 96 + 8 * r:97 + 8 * r])
        inv = 1.0 / jnp.maximum(cnt, 1.0)
        s = (a_ref[0, :, 16 * r:16 * r + _O]
             + a_ref[1, :, 16 * r:16 * r + _O])
        acc = acc + s * (comp_ref[r, 0] * inv)
    z = acc - jnp.max(acc, axis=1, keepdims=True)
    ez = jnp.exp(z)
    o_ref[...] = ez / jnp.sum(ez, axis=1, keepdims=True)


def kernel(x, edge_index, edge_type, basis1, comp1, root1, bias1,
           basis2, comp2, root2, bias2):
    ei3 = edge_index.astype(jnp.int32).reshape(2, _ER, _K)
    t2 = edge_type.astype(jnp.int32).reshape(_ER, _K)

    erows = _ERP // 10
    s4c, s16, x1, x2, xc0, xc1, xc2 = pl.pallas_call(
        _edge_prep_body,
        grid=(10,),
        in_specs=[
            pl.BlockSpec((2, erows, _K), lambda i: (0, i, 0)),
            pl.BlockSpec((erows, _K), lambda i: (i, 0)),
        ],
        out_specs=[
            pl.BlockSpec((2, erows, _K), lambda i: (0, i, 0)),
        ] + [pl.BlockSpec((erows, _K), lambda i: (i, 0))] * 6,
        out_shape=[
            jax.ShapeDtypeStruct((2, _ERP, _K), jnp.int32),
        ] + [jax.ShapeDtypeStruct((_ERP, _K), jnp.int32)] * 6,
    )(ei3, t2)

    w1cat = jnp.concatenate([basis1[0], root1], axis=1)
    b1cat = jnp.concatenate([jnp.zeros((_H,), jnp.float32), bias1])[None, :]
    y1r = pl.pallas_call(
        _mm_body,
        grid=(10,),
        in_specs=[
            pl.BlockSpec((1000, _D), lambda i: (i, 0)),
            pl.BlockSpec((_D, _D), lambda i: (0, 0)),
            pl.BlockSpec((1, _D), lambda i: (0, 0)),
        ],
        out_specs=pl.BlockSpec((1000, _D), lambda i: (i, 0)),
        out_shape=jax.ShapeDtypeStruct((_N, _D), jnp.float32),
    )(x, w1cat, b1cat)

    za1 = jnp.zeros((_ST1, _HW), jnp.float32)
    lane = jnp.arange(_HW)[None, :]
    rel = (jnp.arange(3 * _K) // _K)[:, None]
    pat = ((lane >= 8 * rel) & (lane < 8 * rel + 8)).astype(jnp.float32)
    a1p = _get_seg1()(s4c, x1, xc0, xc1, xc2,
                      y1r.reshape(4 * _N, _HW), za1, pat)
    a1v = a1p.reshape(2, _SA1 // 4, _D)

    w2cat = jnp.concatenate([basis2[0], root2], axis=1)
    b2cat = jnp.concatenate([jnp.zeros((_O,), jnp.float32), bias2])[None, :]
    y2p = pl.pallas_call(
        _comb1_body,
        grid=(_NB,),
        in_specs=[
            pl.BlockSpec((_BLK, _D), lambda i: (i, 0)),
            pl.BlockSpec((2, _BLK, _D), lambda i: (0, i, 0)),
            pl.BlockSpec((_R, 1), lambda i: (0, 0)),
            pl.BlockSpec((_H, _W2), lambda i: (0, 0)),
            pl.BlockSpec((1, _W2), lambda i: (0, 0)),
        ],
        out_specs=pl.BlockSpec((_BLK, _D), lambda i: (i, 0)),
        out_shape=jax.ShapeDtypeStruct((_N, _D), jnp.float32),
    )(y1r, a1v, comp1, w2cat, b2cat)

    za2 = jnp.zeros((_ST2, _L2W), jnp.float32)
    a2p = _get_seg2()(s16, x2, y2p.reshape(8 * _N, _L2W), za2)
    a2v = a2p.reshape(2, _SA2 // 8, _D)

    out = pl.pallas_call(
        _comb2_body,
        grid=(_NB,),
        in_specs=[
            pl.BlockSpec((_BLK, _D), lambda i: (i, 0)),
            pl.BlockSpec((2, _BLK, _D), lambda i: (0, i, 0)),
            pl.BlockSpec((2, _BLK, _D), lambda i: (0, i, 0)),
            pl.BlockSpec((_R, 1), lambda i: (0, 0)),
        ],
        out_specs=pl.BlockSpec((_BLK, _O), lambda i: (i, 0)),
        out_shape=jax.ShapeDtypeStruct((_N, _O), jnp.float32),
    )(y2p, a2v, a1v, comp2)

    return out
